# initial kernel scaffold (unmeasured)
import jax
import jax.numpy as jnp
from jax import lax
from jax.experimental import pallas as pl
from jax.experimental.pallas import tpu as pltpu

M = 4096
N = 8192
K = 4096
HALF = M // 2

BM, BN, BK = 512, 512, 512


def _matmul_body(a_ref, b_ref, o_ref):
    k = pl.program_id(2)

    @pl.when(k == 0)
    def _():
        o_ref[...] = jnp.zeros_like(o_ref)

    o_ref[...] += lax.dot_general(
        a_ref[...],
        b_ref[...],
        dimension_numbers=(((0,), (0,)), ((), ())),
        preferred_element_type=jnp.float32,
    )


def _matmul(x, dy):
    return pl.pallas_call(
        _matmul_body,
        grid=(M // BM, N // BN, K // BK),
        in_specs=[
            pl.BlockSpec((BK, BM), lambda m, n, k: (k, m)),
            pl.BlockSpec((BK, BN), lambda m, n, k: (k, n)),
        ],
        out_specs=pl.BlockSpec((BM, BN), lambda m, n, k: (m, n)),
        out_shape=jax.ShapeDtypeStruct((M, N), jnp.float32),
    )(x, dy)


def _exchange_body(p_ref, out_ref, send_sem, recv_sem):
    my_x = lax.axis_index("x")
    my_y = lax.axis_index("y")
    my_z = lax.axis_index("z")
    partner = (1 - my_x, my_y, my_z)

    barrier = pltpu.get_barrier_semaphore()
    pl.semaphore_signal(
        barrier, inc=1, device_id=partner, device_id_type=pl.DeviceIdType.MESH
    )
    pl.semaphore_wait(barrier, 1)

    other_start = (1 - my_x) * HALF
    rdma = pltpu.make_async_remote_copy(
        src_ref=p_ref.at[pl.ds(other_start, HALF)],
        dst_ref=out_ref,
        send_sem=send_sem,
        recv_sem=recv_sem,
        device_id=partner,
        device_id_type=pl.DeviceIdType.MESH,
    )
    rdma.start()
    rdma.wait()


def _exchange(p_all):
    return pl.pallas_call(
        _exchange_body,
        out_shape=jax.ShapeDtypeStruct((HALF, N), jnp.float32),
        in_specs=[pl.BlockSpec(memory_space=pltpu.ANY)],
        out_specs=pl.BlockSpec(memory_space=pltpu.ANY),
        scratch_shapes=[pltpu.SemaphoreType.DMA, pltpu.SemaphoreType.DMA],
        compiler_params=pltpu.CompilerParams(collective_id=0),
    )(p_all)


def _add_body(a_ref, b_ref, o_ref):
    o_ref[...] = a_ref[...] + b_ref[...]


def _add(a, b):
    bm = 256
    return pl.pallas_call(
        _add_body,
        grid=(HALF // bm,),
        in_specs=[
            pl.BlockSpec((bm, N), lambda i: (i, 0)),
            pl.BlockSpec((bm, N), lambda i: (i, 0)),
        ],
        out_specs=pl.BlockSpec((bm, N), lambda i: (i, 0)),
        out_shape=jax.ShapeDtypeStruct((HALF, N), jnp.float32),
    )(a, b)


def kernel(x, dy):
    my_x = lax.axis_index("x")
    p_all = _matmul(x, dy)
    recv = _exchange(p_all)
    p_mine = lax.dynamic_slice(p_all, (my_x * HALF, 0), (HALF, N))
    return _add(p_mine, recv)


# baseline (device time: 2047703 ns/iter reference)
import jax
import jax.numpy as jnp
from jax import lax
from jax.experimental import pallas as pl
from jax.experimental.pallas import tpu as pltpu

M = 4096
N = 8192
K = 4096
HALF = M // 2

BM, BN, BK = 512, 512, 512


def _matmul_body(a_ref, b_ref, o_ref):
    k = pl.program_id(2)

    @pl.when(k == 0)
    def _():
        o_ref[...] = jnp.zeros_like(o_ref)

    o_ref[...] += lax.dot_general(
        a_ref[...],
        b_ref[...],
        dimension_numbers=(((0,), (0,)), ((), ())),
        preferred_element_type=jnp.float32,
    )


def _matmul(x, dy):
    return pl.pallas_call(
        _matmul_body,
        grid=(M // BM, N // BN, K // BK),
        in_specs=[
            pl.BlockSpec((BK, BM), lambda m, n, k: (k, m)),
            pl.BlockSpec((BK, BN), lambda m, n, k: (k, n)),
        ],
        out_specs=pl.BlockSpec((BM, BN), lambda m, n, k: (m, n)),
        out_shape=jax.ShapeDtypeStruct((M, N), jnp.float32),
    )(x, dy)


def _exchange_body(p_ref, out_ref, send_sem, recv_sem):
    my_x = lax.axis_index("x")
    my_y = lax.axis_index("y")
    my_z = lax.axis_index("z")
    partner = (1 - my_x, my_y, my_z)

    barrier = pltpu.get_barrier_semaphore()
    pl.semaphore_signal(
        barrier, inc=1, device_id=partner, device_id_type=pl.DeviceIdType.MESH
    )
    pl.semaphore_wait(barrier, 1)

    other_start = (1 - my_x) * HALF
    rdma = pltpu.make_async_remote_copy(
        src_ref=p_ref.at[pl.ds(other_start, HALF)],
        dst_ref=out_ref,
        send_sem=send_sem,
        recv_sem=recv_sem,
        device_id=partner,
        device_id_type=pl.DeviceIdType.MESH,
    )
    rdma.start()
    rdma.wait()


def _exchange(p_all):
    return pl.pallas_call(
        _exchange_body,
        out_shape=jax.ShapeDtypeStruct((HALF, N), jnp.float32),
        in_specs=[pl.BlockSpec(memory_space=pl.ANY)],
        out_specs=pl.BlockSpec(memory_space=pl.ANY),
        scratch_shapes=[pltpu.SemaphoreType.DMA, pltpu.SemaphoreType.DMA],
        compiler_params=pltpu.CompilerParams(collective_id=0),
    )(p_all)


def _add_body(a_ref, b_ref, o_ref):
    o_ref[...] = a_ref[...] + b_ref[...]


def _add(a, b):
    bm = 128
    return pl.pallas_call(
        _add_body,
        grid=(HALF // bm,),
        in_specs=[
            pl.BlockSpec((bm, N), lambda i: (i, 0)),
            pl.BlockSpec((bm, N), lambda i: (i, 0)),
        ],
        out_specs=pl.BlockSpec((bm, N), lambda i: (i, 0)),
        out_shape=jax.ShapeDtypeStruct((HALF, N), jnp.float32),
    )(a, b)


def kernel(x, dy):
    my_x = lax.axis_index("x")
    p_all = _matmul(x, dy)
    recv = _exchange(p_all)
    p_mine = lax.dynamic_slice(p_all, (my_x * HALF, 0), (HALF, N))
    return _add(p_mine, recv)


# device time: 904031 ns/iter; 2.2651x vs baseline; 2.2651x over previous
import numpy as np
import jax
import jax.numpy as jnp
from jax import lax
from jax.experimental import pallas as pl
from jax.experimental.pallas import tpu as pltpu

M = 4096
N = 8192
K = 4096
HALF = M // 2
NGRP = 16
BCOL = N // NGRP
NHOP = NGRP - 1

BM, BN, BK = 512, 512, 512

_CYCLE = np.array(
    [
        [0, 0], [0, 1], [0, 2], [0, 3],
        [1, 3], [1, 2], [1, 1],
        [2, 1], [2, 2], [2, 3],
        [3, 3], [3, 2], [3, 1], [3, 0],
        [2, 0], [1, 0],
    ],
    dtype=np.int32,
)
_RIDX = np.zeros(NGRP, dtype=np.int32)
for _p, (_y, _z) in enumerate(_CYCLE):
    _RIDX[_y * 4 + _z] = _p


def _matmul_body(a_ref, b_ref, o_ref):
    k = pl.program_id(2)

    @pl.when(k == 0)
    def _():
        o_ref[...] = jnp.zeros_like(o_ref)

    o_ref[...] += lax.dot_general(
        a_ref[...],
        b_ref[...],
        dimension_numbers=(((0,), (0,)), ((), ())),
        preferred_element_type=jnp.float32,
    )


def _matmul(x, dy_cols):
    return pl.pallas_call(
        _matmul_body,
        grid=(M // BM, BCOL // BN, K // BK),
        in_specs=[
            pl.BlockSpec((BK, BM), lambda m, n, k: (k, m)),
            pl.BlockSpec((BK, BN), lambda m, n, k: (k, n)),
        ],
        out_specs=pl.BlockSpec((BM, BN), lambda m, n, k: (m, n)),
        out_shape=jax.ShapeDtypeStruct((M, BCOL), jnp.float32),
    )(x, dy_cols)


def _comm_body(
    sched_ref,
    p_ref,
    out_ref,
    mine,
    recvx,
    comm,
    send_sems,
    recv_sems,
    xs_sem,
    xr_sem,
    load_sem,
    store_sem,
    credit_sem,
):
    my_x = lax.axis_index("x")
    my_y = lax.axis_index("y")
    my_z = lax.axis_index("z")
    partner = (1 - my_x, my_y, my_z)
    right = (my_x, sched_ref[0], sched_ref[1])
    left = (my_x, sched_ref[2], sched_ref[3])
    r = my_y * 4 + my_z

    barrier = pltpu.get_barrier_semaphore()
    for nbr in (partner, left, right):
        pl.semaphore_signal(
            barrier, inc=1, device_id=nbr, device_id_type=pl.DeviceIdType.MESH
        )
    pl.semaphore_wait(barrier, 3)

    xrdma = pltpu.make_async_remote_copy(
        src_ref=p_ref.at[pl.ds((1 - my_x) * HALF, HALF)],
        dst_ref=recvx,
        send_sem=xs_sem,
        recv_sem=xr_sem,
        device_id=partner,
        device_id_type=pl.DeviceIdType.MESH,
    )
    xrdma.start()
    load = pltpu.make_async_copy(
        p_ref.at[pl.ds(my_x * HALF, HALF)], mine, load_sem
    )
    load.start()
    load.wait()
    xrdma.wait()

    comm[0] = mine[...] + recvx[...]
    st = pltpu.make_async_copy(
        comm.at[0], out_ref.at[:, pl.ds(r * BCOL, BCOL)], store_sem
    )
    st.start()
    st.wait()

    for h in range(NHOP):
        if h >= 1:
            pl.semaphore_wait(credit_sem, 1)
        rdma = pltpu.make_async_remote_copy(
            src_ref=comm.at[h % 2],
            dst_ref=comm.at[(h + 1) % 2],
            send_sem=send_sems.at[h % 2],
            recv_sem=recv_sems.at[(h + 1) % 2],
            device_id=right,
            device_id_type=pl.DeviceIdType.MESH,
        )
        rdma.start()
        rdma.wait()
        org = sched_ref[4 + h]
        st = pltpu.make_async_copy(
            comm.at[(h + 1) % 2],
            out_ref.at[:, pl.ds(org * BCOL, BCOL)],
            store_sem,
        )
        st.start()
        st.wait()
        if h <= NHOP - 2:
            pl.semaphore_signal(
                credit_sem,
                inc=1,
                device_id=left,
                device_id_type=pl.DeviceIdType.MESH,
            )


def _comm(p, sched):
    return pl.pallas_call(
        _comm_body,
        out_shape=jax.ShapeDtypeStruct((HALF, N), jnp.float32),
        in_specs=[
            pl.BlockSpec(memory_space=pltpu.SMEM),
            pl.BlockSpec(memory_space=pl.ANY),
        ],
        out_specs=pl.BlockSpec(memory_space=pl.ANY),
        scratch_shapes=[
            pltpu.VMEM((HALF, BCOL), jnp.float32),
            pltpu.VMEM((HALF, BCOL), jnp.float32),
            pltpu.VMEM((2, HALF, BCOL), jnp.float32),
            pltpu.SemaphoreType.DMA((2,)),
            pltpu.SemaphoreType.DMA((2,)),
            pltpu.SemaphoreType.DMA,
            pltpu.SemaphoreType.DMA,
            pltpu.SemaphoreType.DMA,
            pltpu.SemaphoreType.DMA,
            pltpu.SemaphoreType.REGULAR,
        ],
        compiler_params=pltpu.CompilerParams(collective_id=0),
    )(sched, p)


def kernel(x, dy):
    my_y = lax.axis_index("y")
    my_z = lax.axis_index("z")
    r = my_y * 4 + my_z

    dy_cols = lax.dynamic_slice(dy, (0, r * BCOL), (K, BCOL))
    p = _matmul(x, dy_cols)

    cycle = jnp.asarray(_CYCLE)
    ridx = jnp.asarray(_RIDX)[r]
    right_yz = cycle[(ridx + 1) % NGRP]
    left_yz = cycle[(ridx - 1) % NGRP]
    orig_pos = (ridx - 1 - jnp.arange(NHOP)) % NGRP
    orig_yz = cycle[orig_pos]
    org_blocks = orig_yz[:, 0] * 4 + orig_yz[:, 1]
    sched = jnp.concatenate([right_yz, left_yz, org_blocks]).astype(jnp.int32)

    return _comm(p, sched)


# device time: 569456 ns/iter; 3.5959x vs baseline; 1.5875x over previous
import numpy as np
import jax
import jax.numpy as jnp
from jax import lax
from jax.experimental import pallas as pl
from jax.experimental.pallas import tpu as pltpu

M = 4096
N = 8192
K = 4096
HALF = M // 2
NGRP = 16
BCOL = N // NGRP
NCW = 8
NCCW = 7

BM, BN, BK = 512, 512, 512

_CYCLE = np.array(
    [
        [0, 0], [0, 1], [0, 2], [0, 3],
        [1, 3], [1, 2], [1, 1],
        [2, 1], [2, 2], [2, 3],
        [3, 3], [3, 2], [3, 1], [3, 0],
        [2, 0], [1, 0],
    ],
    dtype=np.int32,
)
_RIDX = np.zeros(NGRP, dtype=np.int32)
for _p, (_y, _z) in enumerate(_CYCLE):
    _RIDX[_y * 4 + _z] = _p


def _matmul_body(a_ref, b_ref, o_ref):
    k = pl.program_id(2)

    @pl.when(k == 0)
    def _():
        o_ref[...] = jnp.zeros_like(o_ref)

    o_ref[...] += lax.dot_general(
        a_ref[...],
        b_ref[...],
        dimension_numbers=(((0,), (0,)), ((), ())),
        preferred_element_type=jnp.float32,
    )


def _matmul(x, dy_cols):
    return pl.pallas_call(
        _matmul_body,
        grid=(M // BM, BCOL // BN, K // BK),
        in_specs=[
            pl.BlockSpec((BK, BM), lambda m, n, k: (k, m)),
            pl.BlockSpec((BK, BN), lambda m, n, k: (k, n)),
        ],
        out_specs=pl.BlockSpec((BM, BN), lambda m, n, k: (m, n)),
        out_shape=jax.ShapeDtypeStruct((M, BCOL), jnp.float32),
    )(x, dy_cols)


def _comm_body(
    sched_ref,
    p_ref,
    out_ref,
    mine,
    recvx,
    ccw_buf,
    cw_buf,
    cw_send_sems,
    cw_recv_sems,
    ccw_send_sems,
    ccw_recv_sems,
    xs_sem,
    xr_sem,
    load_sem,
    store_sem,
    cw_credit,
    ccw_credit,
):
    my_x = lax.axis_index("x")
    my_y = lax.axis_index("y")
    my_z = lax.axis_index("z")
    partner = (1 - my_x, my_y, my_z)
    right = (my_x, sched_ref[0], sched_ref[1])
    left = (my_x, sched_ref[2], sched_ref[3])
    r = my_y * 4 + my_z

    barrier = pltpu.get_barrier_semaphore()
    for nbr in (partner, left, right):
        pl.semaphore_signal(
            barrier, inc=1, device_id=nbr, device_id_type=pl.DeviceIdType.MESH
        )
    pl.semaphore_wait(barrier, 3)

    xrdma = pltpu.make_async_remote_copy(
        src_ref=p_ref.at[pl.ds((1 - my_x) * HALF, HALF)],
        dst_ref=recvx,
        send_sem=xs_sem,
        recv_sem=xr_sem,
        device_id=partner,
        device_id_type=pl.DeviceIdType.MESH,
    )
    xrdma.start()
    load = pltpu.make_async_copy(
        p_ref.at[pl.ds(my_x * HALF, HALF)], mine, load_sem
    )
    load.start()
    load.wait()
    xrdma.wait()

    cw_buf[0] = mine[...] + recvx[...]
    ccw_buf[0] = cw_buf[0]
    st = pltpu.make_async_copy(
        cw_buf.at[0], out_ref.at[:, pl.ds(r * BCOL, BCOL)], store_sem
    )
    st.start()
    st.wait()

    for h in range(NCW):
        if 1 <= h:
            pl.semaphore_wait(cw_credit, 1)
        cw = pltpu.make_async_remote_copy(
            src_ref=cw_buf.at[h % 2],
            dst_ref=cw_buf.at[(h + 1) % 2],
            send_sem=cw_send_sems.at[h % 2],
            recv_sem=cw_recv_sems.at[(h + 1) % 2],
            device_id=right,
            device_id_type=pl.DeviceIdType.MESH,
        )
        cw.start()
        ccw = None
        if h < NCCW:
            if 1 <= h:
                pl.semaphore_wait(ccw_credit, 1)
            ccw = pltpu.make_async_remote_copy(
                src_ref=ccw_buf.at[h % 2],
                dst_ref=ccw_buf.at[(h + 1) % 2],
                send_sem=ccw_send_sems.at[h % 2],
                recv_sem=ccw_recv_sems.at[(h + 1) % 2],
                device_id=left,
                device_id_type=pl.DeviceIdType.MESH,
            )
            ccw.start()

        cw.wait()
        org = sched_ref[4 + h]
        st = pltpu.make_async_copy(
            cw_buf.at[(h + 1) % 2],
            out_ref.at[:, pl.ds(org * BCOL, BCOL)],
            store_sem,
        )
        st.start()
        st.wait()
        if h <= NCW - 2:
            pl.semaphore_signal(
                cw_credit,
                inc=1,
                device_id=left,
                device_id_type=pl.DeviceIdType.MESH,
            )

        if ccw is not None:
            ccw.wait()
            org = sched_ref[4 + NCW + h]
            st = pltpu.make_async_copy(
                ccw_buf.at[(h + 1) % 2],
                out_ref.at[:, pl.ds(org * BCOL, BCOL)],
                store_sem,
            )
            st.start()
            st.wait()
            if h <= NCCW - 2:
                pl.semaphore_signal(
                    ccw_credit,
                    inc=1,
                    device_id=right,
                    device_id_type=pl.DeviceIdType.MESH,
                )


def _comm(p, sched):
    return pl.pallas_call(
        _comm_body,
        out_shape=jax.ShapeDtypeStruct((HALF, N), jnp.float32),
        in_specs=[
            pl.BlockSpec(memory_space=pltpu.SMEM),
            pl.BlockSpec(memory_space=pl.ANY),
        ],
        out_specs=pl.BlockSpec(memory_space=pl.ANY),
        scratch_shapes=[
            pltpu.VMEM((HALF, BCOL), jnp.float32),
            pltpu.VMEM((HALF, BCOL), jnp.float32),
            pltpu.VMEM((2, HALF, BCOL), jnp.float32),
            pltpu.VMEM((2, HALF, BCOL), jnp.float32),
            pltpu.SemaphoreType.DMA((2,)),
            pltpu.SemaphoreType.DMA((2,)),
            pltpu.SemaphoreType.DMA((2,)),
            pltpu.SemaphoreType.DMA((2,)),
            pltpu.SemaphoreType.DMA,
            pltpu.SemaphoreType.DMA,
            pltpu.SemaphoreType.DMA,
            pltpu.SemaphoreType.DMA,
            pltpu.SemaphoreType.REGULAR,
            pltpu.SemaphoreType.REGULAR,
        ],
        compiler_params=pltpu.CompilerParams(collective_id=0),
    )(sched, p)


def kernel(x, dy):
    my_y = lax.axis_index("y")
    my_z = lax.axis_index("z")
    r = my_y * 4 + my_z

    dy_cols = lax.dynamic_slice(dy, (0, r * BCOL), (K, BCOL))
    p = _matmul(x, dy_cols)

    cycle = jnp.asarray(_CYCLE)
    ridx = jnp.asarray(_RIDX)[r]
    right_yz = cycle[(ridx + 1) % NGRP]
    left_yz = cycle[(ridx - 1) % NGRP]
    cw_pos = (ridx - 1 - jnp.arange(NCW)) % NGRP
    ccw_pos = (ridx + 1 + jnp.arange(NCCW)) % NGRP
    org_pos = jnp.concatenate([cw_pos, ccw_pos])
    org_yz = cycle[org_pos]
    org_blocks = org_yz[:, 0] * 4 + org_yz[:, 1]
    sched = jnp.concatenate([right_yz, left_yz, org_blocks]).astype(jnp.int32)

    return _comm(p, sched)


# device time: 545248 ns/iter; 3.7555x vs baseline; 1.0444x over previous
import numpy as np
import jax
import jax.numpy as jnp
from jax import lax
from jax.experimental import pallas as pl
from jax.experimental.pallas import tpu as pltpu

M = 4096
N = 8192
K = 4096
HALF = M // 2
NGRP = 16
BCOL = N // NGRP
NCW = 8
NCCW = 7

BM, BN, BK = 512, 512, 512

_CYCLE = np.array(
    [
        [0, 0], [0, 1], [0, 2], [0, 3],
        [1, 3], [1, 2], [1, 1],
        [2, 1], [2, 2], [2, 3],
        [3, 3], [3, 2], [3, 1], [3, 0],
        [2, 0], [1, 0],
    ],
    dtype=np.int32,
)
_RIDX = np.zeros(NGRP, dtype=np.int32)
for _p, (_y, _z) in enumerate(_CYCLE):
    _RIDX[_y * 4 + _z] = _p


def _matmul_body(a_ref, b_ref, o_ref):
    k = pl.program_id(2)

    @pl.when(k == 0)
    def _():
        o_ref[...] = jnp.zeros_like(o_ref)

    o_ref[...] += lax.dot_general(
        a_ref[...],
        b_ref[...],
        dimension_numbers=(((0,), (0,)), ((), ())),
        preferred_element_type=jnp.float32,
    )


def _matmul(x, dy_cols):
    return pl.pallas_call(
        _matmul_body,
        grid=(M // BM, BCOL // BN, K // BK),
        in_specs=[
            pl.BlockSpec((BK, BM), lambda m, n, k: (k, m)),
            pl.BlockSpec((BK, BN), lambda m, n, k: (k, n)),
        ],
        out_specs=pl.BlockSpec((BM, BN), lambda m, n, k: (m, n)),
        out_shape=jax.ShapeDtypeStruct((M, BCOL), jnp.float32),
    )(x, dy_cols)


def _comm_body(
    sched_ref,
    p_ref,
    out_ref,
    mine,
    recvx,
    ccw_buf,
    cw_buf,
    cw_send_sems,
    cw_recv_sems,
    ccw_send_sems,
    ccw_recv_sems,
    xs_sem,
    xr_sem,
    load_sem,
    store_sem,
    cw_store_sems,
    ccw_store_sems,
    cw_credit,
    ccw_credit,
):
    my_x = lax.axis_index("x")
    my_y = lax.axis_index("y")
    my_z = lax.axis_index("z")
    partner = (1 - my_x, my_y, my_z)
    right = (my_x, sched_ref[0], sched_ref[1])
    left = (my_x, sched_ref[2], sched_ref[3])
    r = my_y * 4 + my_z

    barrier = pltpu.get_barrier_semaphore()
    for nbr in (partner, left, right):
        pl.semaphore_signal(
            barrier, inc=1, device_id=nbr, device_id_type=pl.DeviceIdType.MESH
        )
    pl.semaphore_wait(barrier, 3)

    xrdma = pltpu.make_async_remote_copy(
        src_ref=p_ref.at[pl.ds((1 - my_x) * HALF, HALF)],
        dst_ref=recvx,
        send_sem=xs_sem,
        recv_sem=xr_sem,
        device_id=partner,
        device_id_type=pl.DeviceIdType.MESH,
    )
    xrdma.start()
    load = pltpu.make_async_copy(
        p_ref.at[pl.ds(my_x * HALF, HALF)], mine, load_sem
    )
    load.start()
    load.wait()
    xrdma.wait()

    cw_buf[0] = mine[...] + recvx[...]
    ccw_buf[0] = cw_buf[0]
    seed_store = pltpu.make_async_copy(
        cw_buf.at[0], out_ref.at[:, pl.ds(r * BCOL, BCOL)], store_sem
    )
    seed_store.start()

    cw_stores = [seed_store, None]
    ccw_stores = [None, None]
    for h in range(NCW):
        if 1 <= h:
            pl.semaphore_wait(cw_credit, 1)
        cw = pltpu.make_async_remote_copy(
            src_ref=cw_buf.at[h % 2],
            dst_ref=cw_buf.at[(h + 1) % 2],
            send_sem=cw_send_sems.at[h % 2],
            recv_sem=cw_recv_sems.at[(h + 1) % 2],
            device_id=right,
            device_id_type=pl.DeviceIdType.MESH,
        )
        cw.start()
        ccw = None
        if h < NCCW:
            if 1 <= h:
                pl.semaphore_wait(ccw_credit, 1)
            ccw = pltpu.make_async_remote_copy(
                src_ref=ccw_buf.at[h % 2],
                dst_ref=ccw_buf.at[(h + 1) % 2],
                send_sem=ccw_send_sems.at[h % 2],
                recv_sem=ccw_recv_sems.at[(h + 1) % 2],
                device_id=left,
                device_id_type=pl.DeviceIdType.MESH,
            )
            ccw.start()

        cw.wait()
        if cw_stores[h % 2] is not None:
            cw_stores[h % 2].wait()
        if h <= NCW - 2:
            pl.semaphore_signal(
                cw_credit,
                inc=1,
                device_id=left,
                device_id_type=pl.DeviceIdType.MESH,
            )
        org = sched_ref[4 + h]
        st = pltpu.make_async_copy(
            cw_buf.at[(h + 1) % 2],
            out_ref.at[:, pl.ds(org * BCOL, BCOL)],
            cw_store_sems.at[h % 2],
        )
        st.start()
        cw_stores[(h + 1) % 2] = st

        if ccw is not None:
            ccw.wait()
            if ccw_stores[h % 2] is not None:
                ccw_stores[h % 2].wait()
            if h <= NCCW - 2:
                pl.semaphore_signal(
                    ccw_credit,
                    inc=1,
                    device_id=right,
                    device_id_type=pl.DeviceIdType.MESH,
                )
            org = sched_ref[4 + NCW + h]
            st = pltpu.make_async_copy(
                ccw_buf.at[(h + 1) % 2],
                out_ref.at[:, pl.ds(org * BCOL, BCOL)],
                ccw_store_sems.at[h % 2],
            )
            st.start()
            ccw_stores[(h + 1) % 2] = st

    if cw_stores[NCW % 2] is not None:
        cw_stores[NCW % 2].wait()
    if ccw_stores[NCCW % 2] is not None:
        ccw_stores[NCCW % 2].wait()


def _comm(p, sched):
    return pl.pallas_call(
        _comm_body,
        out_shape=jax.ShapeDtypeStruct((HALF, N), jnp.float32),
        in_specs=[
            pl.BlockSpec(memory_space=pltpu.SMEM),
            pl.BlockSpec(memory_space=pl.ANY),
        ],
        out_specs=pl.BlockSpec(memory_space=pl.ANY),
        scratch_shapes=[
            pltpu.VMEM((HALF, BCOL), jnp.float32),
            pltpu.VMEM((HALF, BCOL), jnp.float32),
            pltpu.VMEM((2, HALF, BCOL), jnp.float32),
            pltpu.VMEM((2, HALF, BCOL), jnp.float32),
            pltpu.SemaphoreType.DMA((2,)),
            pltpu.SemaphoreType.DMA((2,)),
            pltpu.SemaphoreType.DMA((2,)),
            pltpu.SemaphoreType.DMA((2,)),
            pltpu.SemaphoreType.DMA,
            pltpu.SemaphoreType.DMA,
            pltpu.SemaphoreType.DMA,
            pltpu.SemaphoreType.DMA,
            pltpu.SemaphoreType.DMA((2,)),
            pltpu.SemaphoreType.DMA((2,)),
            pltpu.SemaphoreType.REGULAR,
            pltpu.SemaphoreType.REGULAR,
        ],
        compiler_params=pltpu.CompilerParams(collective_id=0),
    )(sched, p)


def kernel(x, dy):
    my_y = lax.axis_index("y")
    my_z = lax.axis_index("z")
    r = my_y * 4 + my_z

    dy_cols = lax.dynamic_slice(dy, (0, r * BCOL), (K, BCOL))
    p = _matmul(x, dy_cols)

    cycle = jnp.asarray(_CYCLE)
    ridx = jnp.asarray(_RIDX)[r]
    right_yz = cycle[(ridx + 1) % NGRP]
    left_yz = cycle[(ridx - 1) % NGRP]
    cw_pos = (ridx - 1 - jnp.arange(NCW)) % NGRP
    ccw_pos = (ridx + 1 + jnp.arange(NCCW)) % NGRP
    org_pos = jnp.concatenate([cw_pos, ccw_pos])
    org_yz = cycle[org_pos]
    org_blocks = org_yz[:, 0] * 4 + org_yz[:, 1]
    sched = jnp.concatenate([right_yz, left_yz, org_blocks]).astype(jnp.int32)

    return _comm(p, sched)


# device time: 522725 ns/iter; 3.9174x vs baseline; 1.0431x over previous
import numpy as np
import jax
import jax.numpy as jnp
from jax import lax
from jax.experimental import pallas as pl
from jax.experimental.pallas import tpu as pltpu

M = 4096
N = 8192
K = 4096
HALF = M // 2
NGRP = 16
BCOL = N // NGRP
NCW = 8
NCCW = 8
HROW = HALF // 2

BM, BN, BK = 512, 512, 512

_CYCLE = np.array(
    [
        [0, 0], [0, 1], [0, 2], [0, 3],
        [1, 3], [1, 2], [1, 1],
        [2, 1], [2, 2], [2, 3],
        [3, 3], [3, 2], [3, 1], [3, 0],
        [2, 0], [1, 0],
    ],
    dtype=np.int32,
)
_RIDX = np.zeros(NGRP, dtype=np.int32)
for _p, (_y, _z) in enumerate(_CYCLE):
    _RIDX[_y * 4 + _z] = _p


def _matmul_body(a_ref, b_ref, o_ref):
    k = pl.program_id(2)

    @pl.when(k == 0)
    def _():
        o_ref[...] = jnp.zeros_like(o_ref)

    o_ref[...] += lax.dot_general(
        a_ref[...],
        b_ref[...],
        dimension_numbers=(((0,), (0,)), ((), ())),
        preferred_element_type=jnp.float32,
    )


def _matmul(x, dy_cols):
    return pl.pallas_call(
        _matmul_body,
        grid=(M // BM, BCOL // BN, K // BK),
        in_specs=[
            pl.BlockSpec((BK, BM), lambda m, n, k: (k, m)),
            pl.BlockSpec((BK, BN), lambda m, n, k: (k, n)),
        ],
        out_specs=pl.BlockSpec((BM, BN), lambda m, n, k: (m, n)),
        out_shape=jax.ShapeDtypeStruct((M, BCOL), jnp.float32),
    )(x, dy_cols)


def _comm_body(
    sched_ref,
    p_ref,
    out_ref,
    mine,
    recvx,
    ccw_buf,
    cw_buf,
    cw_send_sems,
    cw_recv_sems,
    ccw_send_sems,
    ccw_recv_sems,
    xs_sem,
    xr_sem,
    load_sem,
    store_sem,
    cw_store_sems,
    ccw_store_sems,
    cw_credit,
    ccw_credit,
):
    my_x = lax.axis_index("x")
    my_y = lax.axis_index("y")
    my_z = lax.axis_index("z")
    partner = (1 - my_x, my_y, my_z)
    right = (my_x, sched_ref[0], sched_ref[1])
    left = (my_x, sched_ref[2], sched_ref[3])
    r = my_y * 4 + my_z

    barrier = pltpu.get_barrier_semaphore()
    for nbr in (partner, left, right):
        pl.semaphore_signal(
            barrier, inc=1, device_id=nbr, device_id_type=pl.DeviceIdType.MESH
        )
    pl.semaphore_wait(barrier, 3)

    xrdma = pltpu.make_async_remote_copy(
        src_ref=p_ref.at[pl.ds((1 - my_x) * HALF, HALF)],
        dst_ref=recvx,
        send_sem=xs_sem,
        recv_sem=xr_sem,
        device_id=partner,
        device_id_type=pl.DeviceIdType.MESH,
    )
    xrdma.start()
    load = pltpu.make_async_copy(
        p_ref.at[pl.ds(my_x * HALF, HALF)], mine, load_sem
    )
    load.start()
    load.wait()
    xrdma.wait()

    cw_buf[0] = mine[...] + recvx[...]
    ccw_buf[0] = cw_buf[0]
    seed_store = pltpu.make_async_copy(
        cw_buf.at[0], out_ref.at[:, pl.ds(r * BCOL, BCOL)], store_sem
    )
    seed_store.start()

    cw_stores = [seed_store, None]
    ccw_stores = [None, None]
    for h in range(NCW):
        cw_last = h == NCW - 1
        ccw_last = h == NCCW - 1
        if 1 <= h:
            pl.semaphore_wait(cw_credit, 1)
        cw = pltpu.make_async_remote_copy(
            src_ref=(
                cw_buf.at[h % 2, pl.ds(0, HROW)] if cw_last
                else cw_buf.at[h % 2]
            ),
            dst_ref=(
                cw_buf.at[(h + 1) % 2, pl.ds(0, HROW)] if cw_last
                else cw_buf.at[(h + 1) % 2]
            ),
            send_sem=cw_send_sems.at[h % 2],
            recv_sem=cw_recv_sems.at[(h + 1) % 2],
            device_id=right,
            device_id_type=pl.DeviceIdType.MESH,
        )
        cw.start()
        if 1 <= h:
            pl.semaphore_wait(ccw_credit, 1)
        ccw = pltpu.make_async_remote_copy(
            src_ref=(
                ccw_buf.at[h % 2, pl.ds(HROW, HROW)] if ccw_last
                else ccw_buf.at[h % 2]
            ),
            dst_ref=(
                ccw_buf.at[(h + 1) % 2, pl.ds(HROW, HROW)] if ccw_last
                else ccw_buf.at[(h + 1) % 2]
            ),
            send_sem=ccw_send_sems.at[h % 2],
            recv_sem=ccw_recv_sems.at[(h + 1) % 2],
            device_id=left,
            device_id_type=pl.DeviceIdType.MESH,
        )
        ccw.start()

        cw.wait()
        if cw_stores[h % 2] is not None:
            cw_stores[h % 2].wait()
        if h <= NCW - 2:
            pl.semaphore_signal(
                cw_credit,
                inc=1,
                device_id=left,
                device_id_type=pl.DeviceIdType.MESH,
            )
        org = sched_ref[4 + h]
        st = pltpu.make_async_copy(
            (
                cw_buf.at[(h + 1) % 2, pl.ds(0, HROW)] if cw_last
                else cw_buf.at[(h + 1) % 2]
            ),
            (
                out_ref.at[pl.ds(0, HROW), pl.ds(org * BCOL, BCOL)] if cw_last
                else out_ref.at[:, pl.ds(org * BCOL, BCOL)]
            ),
            cw_store_sems.at[h % 2],
        )
        st.start()
        cw_stores[(h + 1) % 2] = st

        ccw.wait()
        if ccw_stores[h % 2] is not None:
            ccw_stores[h % 2].wait()
        if h <= NCCW - 2:
            pl.semaphore_signal(
                ccw_credit,
                inc=1,
                device_id=right,
                device_id_type=pl.DeviceIdType.MESH,
            )
        org = sched_ref[4 + NCW + h]
        st = pltpu.make_async_copy(
            (
                ccw_buf.at[(h + 1) % 2, pl.ds(HROW, HROW)] if ccw_last
                else ccw_buf.at[(h + 1) % 2]
            ),
            (
                out_ref.at[pl.ds(HROW, HROW), pl.ds(org * BCOL, BCOL)]
                if ccw_last
                else out_ref.at[:, pl.ds(org * BCOL, BCOL)]
            ),
            ccw_store_sems.at[h % 2],
        )
        st.start()
        ccw_stores[(h + 1) % 2] = st

    if cw_stores[NCW % 2] is not None:
        cw_stores[NCW % 2].wait()
    if ccw_stores[NCCW % 2] is not None:
        ccw_stores[NCCW % 2].wait()


def _comm(p, sched):
    return pl.pallas_call(
        _comm_body,
        out_shape=jax.ShapeDtypeStruct((HALF, N), jnp.float32),
        in_specs=[
            pl.BlockSpec(memory_space=pltpu.SMEM),
            pl.BlockSpec(memory_space=pl.ANY),
        ],
        out_specs=pl.BlockSpec(memory_space=pl.ANY),
        scratch_shapes=[
            pltpu.VMEM((HALF, BCOL), jnp.float32),
            pltpu.VMEM((HALF, BCOL), jnp.float32),
            pltpu.VMEM((2, HALF, BCOL), jnp.float32),
            pltpu.VMEM((2, HALF, BCOL), jnp.float32),
            pltpu.SemaphoreType.DMA((2,)),
            pltpu.SemaphoreType.DMA((2,)),
            pltpu.SemaphoreType.DMA((2,)),
            pltpu.SemaphoreType.DMA((2,)),
            pltpu.SemaphoreType.DMA,
            pltpu.SemaphoreType.DMA,
            pltpu.SemaphoreType.DMA,
            pltpu.SemaphoreType.DMA,
            pltpu.SemaphoreType.DMA((2,)),
            pltpu.SemaphoreType.DMA((2,)),
            pltpu.SemaphoreType.REGULAR,
            pltpu.SemaphoreType.REGULAR,
        ],
        compiler_params=pltpu.CompilerParams(collective_id=0),
    )(sched, p)


def kernel(x, dy):
    my_y = lax.axis_index("y")
    my_z = lax.axis_index("z")
    r = my_y * 4 + my_z

    dy_cols = lax.dynamic_slice(dy, (0, r * BCOL), (K, BCOL))
    p = _matmul(x, dy_cols)

    cycle = jnp.asarray(_CYCLE)
    ridx = jnp.asarray(_RIDX)[r]
    right_yz = cycle[(ridx + 1) % NGRP]
    left_yz = cycle[(ridx - 1) % NGRP]
    cw_pos = (ridx - 1 - jnp.arange(NCW)) % NGRP
    ccw_pos = (ridx + 1 + jnp.arange(NCCW)) % NGRP
    org_pos = jnp.concatenate([cw_pos, ccw_pos])
    org_yz = cycle[org_pos]
    org_blocks = org_yz[:, 0] * 4 + org_yz[:, 1]
    sched = jnp.concatenate([right_yz, left_yz, org_blocks]).astype(jnp.int32)

    return _comm(p, sched)
